# Initial kernel scaffold; baseline (speedup 1.0000x reference)
import jax
import jax.numpy as jnp
from jax.experimental import pallas as pl


def kernel(token_ids, weight):
    raise NotImplementedError("write your pallas kernel here")


# speedup vs baseline: 3.4609x; 3.4609x over previous
import functools

import jax
import jax.numpy as jnp
from jax import lax
from jax.experimental import pallas as pl
from jax.experimental.pallas import tpu as pltpu
from jax.experimental.pallas import tpu_sc as plsc

_B, _S, _D = 16384, 50, 128
_N = _B * _S
_NC, _NS = 2, 16
_NW = _NC * _NS
_PER_W = _N // _NW
_CH = 128
_NCH = _PER_W // _CH


def _emb_body(ids_hbm, table_hbm, out_hbm, idx_v, buf0, buf1, gsem0, gsem1):
    wid = lax.axis_index("s") * _NC + lax.axis_index("c")
    row0 = wid * _PER_W

    pltpu.sync_copy(ids_hbm.at[wid], idx_v)

    pltpu.async_copy(table_hbm.at[idx_v.at[0]], buf0, gsem0)
    pltpu.async_copy(table_hbm.at[idx_v.at[1]], buf1, gsem1)

    def body(i, carry):
        j0 = 2 * i
        pltpu.make_async_copy(table_hbm.at[idx_v.at[j0]], buf0, gsem0).wait()
        pltpu.sync_copy(buf0, out_hbm.at[pl.ds(row0 + j0 * _CH, _CH)])
        pltpu.async_copy(table_hbm.at[idx_v.at[j0 + 2]], buf0, gsem0)
        pltpu.make_async_copy(table_hbm.at[idx_v.at[j0 + 1]], buf1, gsem1).wait()
        pltpu.sync_copy(buf1, out_hbm.at[pl.ds(row0 + (j0 + 1) * _CH, _CH)])
        pltpu.async_copy(table_hbm.at[idx_v.at[j0 + 3]], buf1, gsem1)
        return carry

    lax.fori_loop(0, _NCH // 2 - 1, body, 0)

    jt = _NCH - 2
    pltpu.make_async_copy(table_hbm.at[idx_v.at[jt]], buf0, gsem0).wait()
    pltpu.sync_copy(buf0, out_hbm.at[pl.ds(row0 + jt * _CH, _CH)])
    pltpu.make_async_copy(table_hbm.at[idx_v.at[jt + 1]], buf1, gsem1).wait()
    pltpu.sync_copy(buf1, out_hbm.at[pl.ds(row0 + (jt + 1) * _CH, _CH)])


@jax.jit
def kernel(token_ids, weight):
    ids = token_ids.reshape(_NW, _NCH, _CH).astype(jnp.int32)
    mesh = plsc.VectorSubcoreMesh(core_axis_name="c", subcore_axis_name="s")
    out = pl.kernel(
        _emb_body,
        mesh=mesh,
        out_type=jax.ShapeDtypeStruct((_N, _D), jnp.float32),
        scratch_types=[
            pltpu.VMEM((_NCH, _CH), jnp.int32),
            pltpu.VMEM((_CH, _D), jnp.float32),
            pltpu.VMEM((_CH, _D), jnp.float32),
            pltpu.SemaphoreType.DMA,
            pltpu.SemaphoreType.DMA,
        ],
    )(ids, weight)
    return out.reshape(_B, _S, _D)
